# baseline (speedup 1.0000x reference)
import functools

import jax
import jax.numpy as jnp
from jax import lax
from jax.experimental import pallas as pl
from jax.experimental.pallas import tpu as pltpu
from jax.experimental.pallas import tpu_sc as plsc

N = 10000
E = 160000
DIM = 3
SEQ = 6
LATENT = 128
EMB = 16
NTYPES = 9
RADIUS = 0.015
VEL_MEAN = 0.0
VEL_STD = 1.0
ACC_MEAN = 0.0
ACC_STD = 1.0
CLAMP = 1.0
LOW = 0.1
HIGH = 0.9

NP = 10240
NW = 32
EB = 128
NB = 40
EP = NW * NB * EB
EPW = NB * EB
RPT = NP // 16

BLK_E = 1024
BLK_N = 1024

_f32 = jnp.float32


def _ln(x, g, b):
    mu = jnp.mean(x, axis=-1, keepdims=True)
    xc = x - mu
    var = jnp.mean(xc * xc, axis=-1, keepdims=True)
    return xc * lax.rsqrt(var + 1e-5) * g + b


def _sc_gather2(A, B, sidx, ridx):
    D = A.shape[1]
    dt = A.dtype
    K, G = (3, 1)
    RG = G * EB
    NGRP = NB // G
    NROUND = NGRP // K + (NGRP % K != 0)
    mesh = plsc.VectorSubcoreMesh(core_axis_name="c", subcore_axis_name="s")

    @functools.partial(
        pl.kernel,
        out_type=(jax.ShapeDtypeStruct((EP, D), dt),) * 2,
        mesh=mesh,
        scratch_types=[
            pltpu.VMEM((NB, EB), jnp.int32),
            pltpu.VMEM((NB, EB), jnp.int32),
            pltpu.VMEM((K, RG, D), dt),
            pltpu.VMEM((K, RG, D), dt),
        ] + [pltpu.SemaphoreType.DMA] * (4 * K),
    )
    def k(a_hbm, b_hbm, si_hbm, ri_hbm, ga_hbm, gb_hbm, si_v, ri_v, bufa, bufb,
          *sems):
        sga = sems[0:K]
        sgb = sems[K:2 * K]
        soa = sems[2 * K:3 * K]
        sob = sems[3 * K:4 * K]
        c = lax.axis_index("c")
        s = lax.axis_index("s")
        wid = s * 2 + c
        base = wid * EPW
        pltpu.sync_copy(si_hbm.at[wid], si_v)
        pltpu.sync_copy(ri_hbm.at[wid], ri_v)

        def fire(gj, b):
            for u in range(G):
                blk = gj * G + u
                pltpu.async_copy(a_hbm.at[si_v.at[blk]],
                                 bufa.at[b].at[pl.ds(u * EB, EB)], sga[b])
                pltpu.async_copy(b_hbm.at[ri_v.at[blk]],
                                 bufb.at[b].at[pl.ds(u * EB, EB)], sgb[b])

        for b in range(K):
            fire(b, b)

        def round_(g, carry):
            for b in range(K):
                gj = g * K + b

                @pl.when(gj < NGRP)
                def _():
                    dst = pl.ds(base + gj * RG, RG)
                    pltpu.make_async_copy(ga_hbm.at[dst], bufa.at[b],
                                          sga[b]).wait()
                    pltpu.make_async_copy(gb_hbm.at[dst], bufb.at[b],
                                          sgb[b]).wait()
                    oa = pltpu.async_copy(bufa.at[b], ga_hbm.at[dst], soa[b])
                    ob = pltpu.async_copy(bufb.at[b], gb_hbm.at[dst], sob[b])
                    oa.wait()
                    ob.wait()

                    @pl.when(gj + K < NGRP)
                    def _():
                        fire(gj + K, b)

            return carry

        lax.fori_loop(0, NROUND, round_, 0)

    return k(A, B, sidx, ridx)


EB_S = 64
NB_S = EPW // EB_S


def _sc_scatter(eu, ridx_s, zeros128):
    mesh = plsc.VectorSubcoreMesh(core_axis_name="c", subcore_axis_name="s")

    K = 4

    @functools.partial(
        pl.kernel,
        out_type=jax.ShapeDtypeStruct((2, NP, LATENT), _f32),
        mesh=mesh,
        scratch_types=[
            pltpu.VMEM((NB_S, EB_S), jnp.int32),
            pltpu.VMEM((K, EB_S, LATENT), _f32),
            pltpu.VMEM_SHARED((NP, LATENT), _f32),
        ] + [pltpu.SemaphoreType.DMA] * (2 * K),
    )
    def k(eu_hbm, ri_hbm, z_hbm, out_hbm, ri_v, buf, acc, *sems):
        sr = sems[0:K]
        ss = sems[K:2 * K]
        c = lax.axis_index("c")
        s = lax.axis_index("s")
        wid = s * 2 + c
        base = wid * EPW
        pltpu.sync_copy(ri_hbm.at[wid], ri_v)
        pltpu.sync_copy(z_hbm, buf.at[0].at[pl.ds(0, EB_S)])

        def zbody(t, carry):
            pltpu.sync_copy(buf.at[0].at[pl.ds(0, EB_S)],
                            acc.at[pl.ds(s * RPT + t * EB_S, EB_S)])
            return carry

        lax.fori_loop(0, RPT // EB_S, zbody, 0)
        plsc.subcore_barrier()

        for b in range(K):
            pltpu.async_copy(eu_hbm.at[pl.ds(base + b * EB_S, EB_S)],
                             buf.at[b], sr[b])

        NG = NB_S // K + (NB_S % K != 0)

        def group(g, carry):
            for b in range(K):
                j = g * K + b

                @pl.when(j < NB_S)
                def _():
                    pltpu.make_async_copy(
                        eu_hbm.at[pl.ds(base + j * EB_S, EB_S)], buf.at[b],
                        sr[b]).wait()
                    sc = pltpu.async_copy(buf.at[b], acc.at[ri_v.at[j]],
                                          ss[b], add=True)
                    sc.wait()

                    @pl.when(j + K < NB_S)
                    def _():
                        pltpu.async_copy(
                            eu_hbm.at[pl.ds(base + (j + K) * EB_S, EB_S)],
                            buf.at[b], sr[b])

            return carry

        lax.fori_loop(0, NG, group, 0)
        plsc.subcore_barrier()

        def obody(t, carry):
            r0 = s * RPT + t * EB_S
            pltpu.sync_copy(acc.at[pl.ds(r0, EB_S)],
                            buf.at[0].at[pl.ds(0, EB_S)])
            pltpu.sync_copy(buf.at[0].at[pl.ds(0, EB_S)],
                            out_hbm.at[c].at[pl.ds(r0, EB_S)])
            return carry

        lax.fori_loop(0, RPT // EB_S, obody, 0)

    return k(eu, ridx_s, zeros128)


def _node_enc_body(pf, tp, w1v, w1dl, w1du, w1t, b1, w2, b2, w3, b3, g, bt,
                   ws0, wr0, x_o, a_o, b_o):
    p = pf[...]
    vel = (p[:, 3:18] - p[:, 0:15] - VEL_MEAN) * (1.0 / VEL_STD)
    last = p[:, 15:18]
    ndl = jnp.clip((last - LOW) * (1.0 / RADIUS), -CLAMP, CLAMP)
    ndu = jnp.clip((HIGH - last) * (1.0 / RADIUS), -CLAMP, CLAMP)
    t = tp[...]
    oh = (t == lax.broadcasted_iota(jnp.int32, (t.shape[0], NTYPES), 1)
          ).astype(_f32)
    h1 = jnp.maximum(vel @ w1v[...] + ndl @ w1dl[...] + ndu @ w1du[...]
                     + oh @ w1t[...] + b1[...], 0.0)
    h2 = jnp.maximum(h1 @ w2[...] + b2[...], 0.0)
    x = _ln(h2 @ w3[...] + b3[...], g[...], bt[...])
    x_o[...] = x
    a_o[...] = x @ ws0[...]
    b_o[...] = x @ wr0[...]


def _edge_enc_body(ps, pr, w1r, w1d, b1, w2, b2, w3, b3, g, bt, e_o):
    rel = (ps[...][:, 0:3] - pr[...][:, 0:3]) * (1.0 / RADIUS)
    dist = jnp.sqrt(jnp.sum(rel * rel, axis=1, keepdims=True))
    h1 = jnp.maximum(rel @ w1r[...] + dist * w1d[...] + b1[...], 0.0)
    h2 = jnp.maximum(h1 @ w2[...] + b2[...], 0.0)
    e_o[...] = _ln(h2 @ w3[...] + b3[...], g[...], bt[...])


def _edge_proc_body(ga, gb, e_in, we, b1, w2, b2, w3, b3, g, bt, eu_o, en_o):
    i = pl.program_id(0)
    e = e_in[...]
    h1 = jnp.maximum(ga[...] + gb[...] + e @ we[...] + b1[...], 0.0)
    h2 = jnp.maximum(h1 @ w2[...] + b2[...], 0.0)
    u = _ln(h2 @ w3[...] + b3[...], g[...], bt[...])
    rows = i * BLK_E + lax.broadcasted_iota(jnp.int32, (BLK_E, 1), 0)
    u = jnp.where(rows < E, u, 0.0)
    eu_o[...] = u
    en_o[...] = e + u


def _node_proc_body(x_in, g0, g1, w1x, w1a, b1, w2, b2, w3, b3, g, bt,
                    wsn, wrn, x_o, a_o, b_o):
    x = x_in[...]
    agg = g0[...] + g1[...]
    h1 = jnp.maximum(x @ w1x[...] + agg @ w1a[...] + b1[...], 0.0)
    h2 = jnp.maximum(h1 @ w2[...] + b2[...], 0.0)
    u = _ln(h2 @ w3[...] + b3[...], g[...], bt[...])
    xn = x + u
    x_o[...] = xn
    a_o[...] = xn @ wsn[...]
    b_o[...] = xn @ wrn[...]


def _dec_body(x_in, l8, p8, w1, b1, w2, b2, w3p, b3p, o_ref):
    h1 = jnp.maximum(x_in[...] @ w1[...] + b1[...], 0.0)
    h2 = jnp.maximum(h1 @ w2[...] + b2[...], 0.0)
    acc8 = (h2 @ w3p[...] + b3p[...]) * ACC_STD + ACC_MEAN
    o_ref[...] = 2.0 * l8[...] - p8[...] + acc8


def _row_spec(blk, d):
    return pl.BlockSpec((blk, d), lambda i: (i, 0))


def _w_spec(d0, d1):
    return pl.BlockSpec((d0, d1), lambda i: (0, 0))


def _node_enc(posflat, types2, w):
    grid = NP // BLK_N
    out = pl.pallas_call(
        _node_enc_body,
        grid=grid,
        in_specs=[
            _row_spec(BLK_N, SEQ * DIM), _row_spec(BLK_N, 1),
            _w_spec(15, LATENT), _w_spec(3, LATENT), _w_spec(3, LATENT),
            _w_spec(NTYPES, LATENT), _w_spec(1, LATENT),
            _w_spec(LATENT, LATENT), _w_spec(1, LATENT),
            _w_spec(LATENT, LATENT), _w_spec(1, LATENT),
            _w_spec(1, LATENT), _w_spec(1, LATENT),
            _w_spec(LATENT, LATENT), _w_spec(LATENT, LATENT),
        ],
        out_specs=[_row_spec(BLK_N, LATENT)] * 3,
        out_shape=[jax.ShapeDtypeStruct((NP, LATENT), _f32)] * 3,
    )(posflat, types2, w['n_w1v'], w['n_w1dl'], w['n_w1du'], w['n_w1t'],
      w['n_b1'], w['n_w2'], w['n_b2'], w['n_w3'], w['n_b3'], w['n_g'],
      w['n_bt'], w['ws0'], w['wr0'])
    return out


def _edge_enc(gps, gpr, w):
    grid = EP // BLK_E
    return pl.pallas_call(
        _edge_enc_body,
        grid=grid,
        in_specs=[
            _row_spec(BLK_E, LATENT), _row_spec(BLK_E, LATENT),
            _w_spec(3, LATENT), _w_spec(1, LATENT), _w_spec(1, LATENT),
            _w_spec(LATENT, LATENT), _w_spec(1, LATENT),
            _w_spec(LATENT, LATENT), _w_spec(1, LATENT),
            _w_spec(1, LATENT), _w_spec(1, LATENT),
        ],
        out_specs=_row_spec(BLK_E, LATENT),
        out_shape=jax.ShapeDtypeStruct((EP, LATENT), _f32),
    )(gps, gpr, w['e_w1r'], w['e_w1d'], w['e_b1'], w['e_w2'], w['e_b2'],
      w['e_w3'], w['e_b3'], w['e_g'], w['e_bt'])


def _edge_proc(gA, gB, e, w):
    grid = EP // BLK_E
    return pl.pallas_call(
        _edge_proc_body,
        grid=grid,
        in_specs=[
            _row_spec(BLK_E, LATENT), _row_spec(BLK_E, LATENT),
            _row_spec(BLK_E, LATENT),
            _w_spec(LATENT, LATENT), _w_spec(1, LATENT),
            _w_spec(LATENT, LATENT), _w_spec(1, LATENT),
            _w_spec(LATENT, LATENT), _w_spec(1, LATENT),
            _w_spec(1, LATENT), _w_spec(1, LATENT),
        ],
        out_specs=[_row_spec(BLK_E, LATENT)] * 2,
        out_shape=[jax.ShapeDtypeStruct((EP, LATENT), _f32)] * 2,
    )(gA, gB, e, w['pe_we'], w['pe_b1'], w['pe_w2'], w['pe_b2'], w['pe_w3'],
      w['pe_b3'], w['pe_g'], w['pe_bt'])


def _node_proc(x, agg, w):
    grid = NP // BLK_N
    a3 = pl.BlockSpec((1, BLK_N, LATENT), lambda i: (0, i, 0))
    b3 = pl.BlockSpec((1, BLK_N, LATENT), lambda i: (1, i, 0))

    def body(x_in, agg_in0, agg_in1, *rest):
        _node_proc_body(x_in, agg_in0.at[0], agg_in1.at[0], *rest)

    return pl.pallas_call(
        body,
        grid=grid,
        in_specs=[
            _row_spec(BLK_N, LATENT), a3, b3,
            _w_spec(LATENT, LATENT), _w_spec(LATENT, LATENT),
            _w_spec(1, LATENT),
            _w_spec(LATENT, LATENT), _w_spec(1, LATENT),
            _w_spec(LATENT, LATENT), _w_spec(1, LATENT),
            _w_spec(1, LATENT), _w_spec(1, LATENT),
            _w_spec(LATENT, LATENT), _w_spec(LATENT, LATENT),
        ],
        out_specs=[_row_spec(BLK_N, LATENT)] * 3,
        out_shape=[jax.ShapeDtypeStruct((NP, LATENT), _f32)] * 3,
    )(x, agg, agg, w['pn_w1x'], w['pn_w1a'], w['pn_b1'], w['pn_w2'],
      w['pn_b2'], w['pn_w3'], w['pn_b3'], w['pn_g'], w['pn_bt'],
      w['wsn'], w['wrn'])


def _decoder(x, l8, p8, w):
    grid = NP // BLK_N
    return pl.pallas_call(
        _dec_body,
        grid=grid,
        in_specs=[
            _row_spec(BLK_N, LATENT), _row_spec(BLK_N, 8), _row_spec(BLK_N, 8),
            _w_spec(LATENT, LATENT), _w_spec(1, LATENT),
            _w_spec(LATENT, LATENT), _w_spec(1, LATENT),
            _w_spec(LATENT, 8), _w_spec(1, 8),
        ],
        out_specs=_row_spec(BLK_N, 8),
        out_shape=jax.ShapeDtypeStruct((NP, 8), _f32),
    )(x, l8, p8, w['d_w1'], w['d_b1'], w['d_w2'], w['d_b2'], w['d_w3p'],
      w['d_b3p'])


def _row(v):
    return v.reshape(1, -1)


def kernel(position_sequence, edge_index, particle_types, params):
    pos = position_sequence.astype(_f32)
    last = pos[:, -1]
    prev = pos[:, -2]

    posflat = jnp.pad(pos.reshape(N, SEQ * DIM), ((0, NP - N), (0, 0)))
    types2 = jnp.pad(particle_types.reshape(N, 1).astype(jnp.int32),
                     ((0, NP - N), (0, 0)))
    sidx = jnp.pad(edge_index[0], (0, EP - E)).reshape(NW, NB, EB)
    ridx = jnp.pad(edge_index[1], (0, EP - E)).reshape(NW, NB, EB)
    ridx_s = jnp.pad(edge_index[1], (0, EP - E)).reshape(NW, NB_S, EB_S)
    pos128 = jnp.pad(last, ((0, NP - N), (0, LATENT - DIM)))
    l8 = jnp.pad(last, ((0, NP - N), (0, 8 - DIM)))
    p8 = jnp.pad(prev, ((0, NP - N), (0, 8 - DIM)))
    zeros128 = jnp.zeros((EB_S, LATENT), _f32)

    ne = params['node_encoder']
    ee = params['edge_encoder']
    de = params['decoder']
    procs = params['processor']

    nW1 = ne['layers'][0][0]
    w = {
        'n_w1v': nW1[0:15], 'n_w1dl': nW1[15:18], 'n_w1du': nW1[18:21],
        'n_w1t': params['type_emb'] @ nW1[21:37],
        'n_b1': _row(ne['layers'][0][1]),
        'n_w2': ne['layers'][1][0], 'n_b2': _row(ne['layers'][1][1]),
        'n_w3': ne['layers'][2][0], 'n_b3': _row(ne['layers'][2][1]),
        'n_g': _row(ne['ln'][0]), 'n_bt': _row(ne['ln'][1]),
        'e_w1r': ee['layers'][0][0][0:3],
        'e_w1d': _row(ee['layers'][0][0][3]),
        'e_b1': _row(ee['layers'][0][1]),
        'e_w2': ee['layers'][1][0], 'e_b2': _row(ee['layers'][1][1]),
        'e_w3': ee['layers'][2][0], 'e_b3': _row(ee['layers'][2][1]),
        'e_g': _row(ee['ln'][0]), 'e_bt': _row(ee['ln'][1]),
        'd_w1': de['layers'][0][0], 'd_b1': _row(de['layers'][0][1]),
        'd_w2': de['layers'][1][0], 'd_b2': _row(de['layers'][1][1]),
        'd_w3p': jnp.pad(de['layers'][2][0], ((0, 0), (0, 8 - DIM))),
        'd_b3p': _row(jnp.pad(de['layers'][2][1], (0, 8 - DIM))),
    }

    def stk(f):
        return jnp.stack([f(p) for p in procs])

    ws_all = stk(lambda p: p['edge']['layers'][0][0][0:LATENT])
    wr_all = stk(lambda p: p['edge']['layers'][0][0][LATENT:2 * LATENT])
    w['ws0'] = ws_all[0]
    w['wr0'] = wr_all[0]

    wstack = {
        'pe_we': stk(lambda p: p['edge']['layers'][0][0][2 * LATENT:]),
        'pe_b1': stk(lambda p: _row(p['edge']['layers'][0][1])),
        'pe_w2': stk(lambda p: p['edge']['layers'][1][0]),
        'pe_b2': stk(lambda p: _row(p['edge']['layers'][1][1])),
        'pe_w3': stk(lambda p: p['edge']['layers'][2][0]),
        'pe_b3': stk(lambda p: _row(p['edge']['layers'][2][1])),
        'pe_g': stk(lambda p: _row(p['edge']['ln'][0])),
        'pe_bt': stk(lambda p: _row(p['edge']['ln'][1])),
        'pn_w1x': stk(lambda p: p['node']['layers'][0][0][0:LATENT]),
        'pn_w1a': stk(lambda p: p['node']['layers'][0][0][LATENT:]),
        'pn_b1': stk(lambda p: _row(p['node']['layers'][0][1])),
        'pn_w2': stk(lambda p: p['node']['layers'][1][0]),
        'pn_b2': stk(lambda p: _row(p['node']['layers'][1][1])),
        'pn_w3': stk(lambda p: p['node']['layers'][2][0]),
        'pn_b3': stk(lambda p: _row(p['node']['layers'][2][1])),
        'pn_g': stk(lambda p: _row(p['node']['ln'][0])),
        'pn_bt': stk(lambda p: _row(p['node']['ln'][1])),
        'wsn': jnp.roll(ws_all, -1, axis=0),
        'wrn': jnp.roll(wr_all, -1, axis=0),
    }

    x0, A0, B0 = _node_enc(posflat, types2, w)
    gps, gpr = _sc_gather2(pos128, pos128, sidx, ridx)
    e0 = _edge_enc(gps, gpr, w)

    def step(carry, ws):
        x, e, A, B = carry
        gA, gB = _sc_gather2(A, B, sidx, ridx)
        eu, en = _edge_proc(gA, gB, e, ws)
        agg = _sc_scatter(eu, ridx_s, zeros128)
        xn, An, Bn = _node_proc(x, agg, ws)
        return (xn, en, An, Bn), None

    (x, _, _, _), _ = lax.scan(step, (x0, e0, A0, B0), wstack)

    out8 = _decoder(x, l8, p8, w)
    return out8[:N, 0:DIM]


# speedup vs baseline: 1.1872x; 1.1872x over previous
import functools

import jax
import jax.numpy as jnp
from jax import lax
from jax.experimental import pallas as pl
from jax.experimental.pallas import tpu as pltpu
from jax.experimental.pallas import tpu_sc as plsc

N = 10000
E = 160000
DIM = 3
SEQ = 6
LATENT = 128
EMB = 16
NTYPES = 9
RADIUS = 0.015
VEL_MEAN = 0.0
VEL_STD = 1.0
ACC_MEAN = 0.0
ACC_STD = 1.0
CLAMP = 1.0
LOW = 0.1
HIGH = 0.9

NP = 10240
NW = 32
EB = 128
NB = 40
EP = NW * NB * EB
EPW = NB * EB
RPT = NP // 16

BLK_E = 1024
BLK_N = 1024

_f32 = jnp.float32


def _ln(x, g, b):
    mu = jnp.mean(x, axis=-1, keepdims=True)
    xc = x - mu
    var = jnp.mean(xc * xc, axis=-1, keepdims=True)
    return xc * lax.rsqrt(var + 1e-5) * g + b


def _sc_gather2(A, B, sidx, ridx):
    D = A.shape[1]
    dt = A.dtype
    K, G = (3, 1)
    RG = G * EB
    NGRP = NB // G
    NROUND = NGRP // K + (NGRP % K != 0)
    mesh = plsc.VectorSubcoreMesh(core_axis_name="c", subcore_axis_name="s")

    @functools.partial(
        pl.kernel,
        out_type=(jax.ShapeDtypeStruct((EP, D), dt),) * 2,
        mesh=mesh,
        scratch_types=[
            pltpu.VMEM((NB, EB), jnp.int32),
            pltpu.VMEM((NB, EB), jnp.int32),
            pltpu.VMEM((K, RG, D), dt),
            pltpu.VMEM((K, RG, D), dt),
        ] + [pltpu.SemaphoreType.DMA] * (4 * K),
    )
    def k(a_hbm, b_hbm, si_hbm, ri_hbm, ga_hbm, gb_hbm, si_v, ri_v, bufa, bufb,
          *sems):
        sga = sems[0:K]
        sgb = sems[K:2 * K]
        soa = sems[2 * K:3 * K]
        sob = sems[3 * K:4 * K]
        c = lax.axis_index("c")
        s = lax.axis_index("s")
        wid = s * 2 + c
        base = wid * EPW
        pltpu.sync_copy(si_hbm.at[wid], si_v)
        pltpu.sync_copy(ri_hbm.at[wid], ri_v)

        def fire(gj, b):
            for u in range(G):
                blk = gj * G + u
                pltpu.async_copy(a_hbm.at[si_v.at[blk]],
                                 bufa.at[b].at[pl.ds(u * EB, EB)], sga[b])
                pltpu.async_copy(b_hbm.at[ri_v.at[blk]],
                                 bufb.at[b].at[pl.ds(u * EB, EB)], sgb[b])

        for b in range(K):
            fire(b, b)

        def round_(g, carry):
            for b in range(K):
                gj = g * K + b

                @pl.when(gj < NGRP)
                def _():
                    dst = pl.ds(base + gj * RG, RG)
                    pltpu.make_async_copy(ga_hbm.at[dst], bufa.at[b],
                                          sga[b]).wait()
                    pltpu.make_async_copy(gb_hbm.at[dst], bufb.at[b],
                                          sgb[b]).wait()
                    oa = pltpu.async_copy(bufa.at[b], ga_hbm.at[dst], soa[b])
                    ob = pltpu.async_copy(bufb.at[b], gb_hbm.at[dst], sob[b])
                    oa.wait()
                    ob.wait()

                    @pl.when(gj + K < NGRP)
                    def _():
                        fire(gj + K, b)

            return carry

        lax.fori_loop(0, NROUND, round_, 0)

    return k(A, B, sidx, ridx)


EB_S = 128
NB_S = EPW // EB_S
SN = 10112
RPT_S = SN // 16
_WCHUNKS = [128, 128, 128, 128, 120]


def _sc_scatter(eu, ridx_s, zeros128):
    mesh = plsc.VectorSubcoreMesh(core_axis_name="c", subcore_axis_name="s")

    NBUF = 3

    @functools.partial(
        pl.kernel,
        out_type=jax.ShapeDtypeStruct((2, NP, LATENT), _f32),
        mesh=mesh,
        scratch_types=[
            pltpu.VMEM((NBUF, EB_S), jnp.int32),
            pltpu.VMEM((NBUF, EB_S, LATENT), _f32),
            pltpu.VMEM_SHARED((SN, LATENT), _f32),
        ] + [pltpu.SemaphoreType.DMA] * (2 * NBUF),
    )
    def k(eu_hbm, ri_hbm, z_hbm, out_hbm, idxv, buf, acc, *sems):
        sr = sems[0:NBUF]
        ss = sems[NBUF:2 * NBUF]
        c = lax.axis_index("c")
        s = lax.axis_index("s")
        wid = s * 2 + c
        base = wid * EPW

        def fire(j, b):
            pltpu.async_copy(eu_hbm.at[pl.ds(base + j * EB_S, EB_S)],
                             buf.at[b], sr[b])
            pltpu.async_copy(ri_hbm.at[wid].at[j], idxv.at[b], sr[b])

        pltpu.sync_copy(z_hbm, buf.at[0])
        r0z = s * RPT_S
        off = 0
        for ch in _WCHUNKS:
            pltpu.sync_copy(buf.at[0].at[pl.ds(0, ch)],
                            acc.at[pl.ds(r0z + off, ch)])
            off += ch
        plsc.subcore_barrier()

        for b in range(2):
            fire(b, b)

        NG = NB_S // NBUF + (NB_S % NBUF != 0)

        def group(g, carry):
            for b in range(NBUF):
                j = g * NBUF + b
                b2 = (b + 2) % NBUF

                @pl.when(j < NB_S)
                def _():
                    pltpu.make_async_copy(
                        eu_hbm.at[pl.ds(base + j * EB_S, EB_S)], buf.at[b],
                        sr[b]).wait()
                    pltpu.make_async_copy(ri_hbm.at[wid].at[j], idxv.at[b],
                                          sr[b]).wait()
                    pltpu.async_copy(buf.at[b], acc.at[idxv.at[b]], ss[b],
                                     add=True)
                    j2 = j + 2

                    @pl.when(j2 < NB_S)
                    def _():
                        @pl.when(j2 >= NBUF)
                        def _():
                            pltpu.make_async_copy(
                                buf.at[b2], acc.at[idxv.at[b2]],
                                ss[b2]).wait()

                        fire(j2, b2)

            return carry

        lax.fori_loop(0, NG, group, 0)
        for b in range(NBUF):
            j = NB_S - NBUF + b
            pltpu.make_async_copy(buf.at[j % NBUF], acc.at[idxv.at[j % NBUF]],
                                  ss[j % NBUF]).wait()
        plsc.subcore_barrier()

        off = 0
        for ch in _WCHUNKS:
            r0 = s * RPT_S + off
            pltpu.sync_copy(acc.at[pl.ds(r0, ch)], buf.at[0].at[pl.ds(0, ch)])
            pltpu.sync_copy(buf.at[0].at[pl.ds(0, ch)],
                            out_hbm.at[c].at[pl.ds(r0, ch)])
            off += ch

    return k(eu, ridx_s, zeros128)


def _node_enc_body(pf, tp, w1v, w1dl, w1du, w1t, b1, w2, b2, w3, b3, g, bt,
                   ws0, wr0, x_o, a_o, b_o):
    p = pf[...]
    vel = (p[:, 3:18] - p[:, 0:15] - VEL_MEAN) * (1.0 / VEL_STD)
    last = p[:, 15:18]
    ndl = jnp.clip((last - LOW) * (1.0 / RADIUS), -CLAMP, CLAMP)
    ndu = jnp.clip((HIGH - last) * (1.0 / RADIUS), -CLAMP, CLAMP)
    t = tp[...]
    oh = (t == lax.broadcasted_iota(jnp.int32, (t.shape[0], NTYPES), 1)
          ).astype(_f32)
    h1 = jnp.maximum(vel @ w1v[...] + ndl @ w1dl[...] + ndu @ w1du[...]
                     + oh @ w1t[...] + b1[...], 0.0)
    h2 = jnp.maximum(h1 @ w2[...] + b2[...], 0.0)
    x = _ln(h2 @ w3[...] + b3[...], g[...], bt[...])
    x_o[...] = x
    a_o[...] = x @ ws0[...]
    b_o[...] = x @ wr0[...]


def _edge_enc_body(ps, pr, w1r, w1d, b1, w2, b2, w3, b3, g, bt, e_o):
    rel = (ps[...][:, 0:3] - pr[...][:, 0:3]) * (1.0 / RADIUS)
    dist = jnp.sqrt(jnp.sum(rel * rel, axis=1, keepdims=True))
    h1 = jnp.maximum(rel @ w1r[...] + dist * w1d[...] + b1[...], 0.0)
    h2 = jnp.maximum(h1 @ w2[...] + b2[...], 0.0)
    e_o[...] = _ln(h2 @ w3[...] + b3[...], g[...], bt[...])


def _edge_proc_body(ga, gb, e_in, we, b1, w2, b2, w3, b3, g, bt, eu_o, en_o):
    i = pl.program_id(0)
    e = e_in[...]
    h1 = jnp.maximum(ga[...] + gb[...] + e @ we[...] + b1[...], 0.0)
    h2 = jnp.maximum(h1 @ w2[...] + b2[...], 0.0)
    u = _ln(h2 @ w3[...] + b3[...], g[...], bt[...])
    rows = i * BLK_E + lax.broadcasted_iota(jnp.int32, (BLK_E, 1), 0)
    u = jnp.where(rows < E, u, 0.0)
    eu_o[...] = u
    en_o[...] = e + u


def _node_proc_body(x_in, g0, g1, w1x, w1a, b1, w2, b2, w3, b3, g, bt,
                    wsn, wrn, x_o, a_o, b_o):
    x = x_in[...]
    agg = g0[...] + g1[...]
    h1 = jnp.maximum(x @ w1x[...] + agg @ w1a[...] + b1[...], 0.0)
    h2 = jnp.maximum(h1 @ w2[...] + b2[...], 0.0)
    u = _ln(h2 @ w3[...] + b3[...], g[...], bt[...])
    xn = x + u
    x_o[...] = xn
    a_o[...] = xn @ wsn[...]
    b_o[...] = xn @ wrn[...]


def _dec_body(x_in, l8, p8, w1, b1, w2, b2, w3p, b3p, o_ref):
    h1 = jnp.maximum(x_in[...] @ w1[...] + b1[...], 0.0)
    h2 = jnp.maximum(h1 @ w2[...] + b2[...], 0.0)
    acc8 = (h2 @ w3p[...] + b3p[...]) * ACC_STD + ACC_MEAN
    o_ref[...] = 2.0 * l8[...] - p8[...] + acc8


def _row_spec(blk, d):
    return pl.BlockSpec((blk, d), lambda i: (i, 0))


def _w_spec(d0, d1):
    return pl.BlockSpec((d0, d1), lambda i: (0, 0))


def _node_enc(posflat, types2, w):
    grid = NP // BLK_N
    out = pl.pallas_call(
        _node_enc_body,
        grid=grid,
        in_specs=[
            _row_spec(BLK_N, SEQ * DIM), _row_spec(BLK_N, 1),
            _w_spec(15, LATENT), _w_spec(3, LATENT), _w_spec(3, LATENT),
            _w_spec(NTYPES, LATENT), _w_spec(1, LATENT),
            _w_spec(LATENT, LATENT), _w_spec(1, LATENT),
            _w_spec(LATENT, LATENT), _w_spec(1, LATENT),
            _w_spec(1, LATENT), _w_spec(1, LATENT),
            _w_spec(LATENT, LATENT), _w_spec(LATENT, LATENT),
        ],
        out_specs=[_row_spec(BLK_N, LATENT)] * 3,
        out_shape=[jax.ShapeDtypeStruct((NP, LATENT), _f32)] * 3,
    )(posflat, types2, w['n_w1v'], w['n_w1dl'], w['n_w1du'], w['n_w1t'],
      w['n_b1'], w['n_w2'], w['n_b2'], w['n_w3'], w['n_b3'], w['n_g'],
      w['n_bt'], w['ws0'], w['wr0'])
    return out


def _edge_enc(gps, gpr, w):
    grid = EP // BLK_E
    return pl.pallas_call(
        _edge_enc_body,
        grid=grid,
        in_specs=[
            _row_spec(BLK_E, LATENT), _row_spec(BLK_E, LATENT),
            _w_spec(3, LATENT), _w_spec(1, LATENT), _w_spec(1, LATENT),
            _w_spec(LATENT, LATENT), _w_spec(1, LATENT),
            _w_spec(LATENT, LATENT), _w_spec(1, LATENT),
            _w_spec(1, LATENT), _w_spec(1, LATENT),
        ],
        out_specs=_row_spec(BLK_E, LATENT),
        out_shape=jax.ShapeDtypeStruct((EP, LATENT), _f32),
    )(gps, gpr, w['e_w1r'], w['e_w1d'], w['e_b1'], w['e_w2'], w['e_b2'],
      w['e_w3'], w['e_b3'], w['e_g'], w['e_bt'])


def _edge_proc(gA, gB, e, w):
    grid = EP // BLK_E
    return pl.pallas_call(
        _edge_proc_body,
        grid=grid,
        in_specs=[
            _row_spec(BLK_E, LATENT), _row_spec(BLK_E, LATENT),
            _row_spec(BLK_E, LATENT),
            _w_spec(LATENT, LATENT), _w_spec(1, LATENT),
            _w_spec(LATENT, LATENT), _w_spec(1, LATENT),
            _w_spec(LATENT, LATENT), _w_spec(1, LATENT),
            _w_spec(1, LATENT), _w_spec(1, LATENT),
        ],
        out_specs=[_row_spec(BLK_E, LATENT)] * 2,
        out_shape=[jax.ShapeDtypeStruct((EP, LATENT), _f32)] * 2,
    )(gA, gB, e, w['pe_we'], w['pe_b1'], w['pe_w2'], w['pe_b2'], w['pe_w3'],
      w['pe_b3'], w['pe_g'], w['pe_bt'])


def _node_proc(x, agg, w):
    grid = NP // BLK_N
    a3 = pl.BlockSpec((1, BLK_N, LATENT), lambda i: (0, i, 0))
    b3 = pl.BlockSpec((1, BLK_N, LATENT), lambda i: (1, i, 0))

    def body(x_in, agg_in0, agg_in1, *rest):
        _node_proc_body(x_in, agg_in0.at[0], agg_in1.at[0], *rest)

    return pl.pallas_call(
        body,
        grid=grid,
        in_specs=[
            _row_spec(BLK_N, LATENT), a3, b3,
            _w_spec(LATENT, LATENT), _w_spec(LATENT, LATENT),
            _w_spec(1, LATENT),
            _w_spec(LATENT, LATENT), _w_spec(1, LATENT),
            _w_spec(LATENT, LATENT), _w_spec(1, LATENT),
            _w_spec(1, LATENT), _w_spec(1, LATENT),
            _w_spec(LATENT, LATENT), _w_spec(LATENT, LATENT),
        ],
        out_specs=[_row_spec(BLK_N, LATENT)] * 3,
        out_shape=[jax.ShapeDtypeStruct((NP, LATENT), _f32)] * 3,
    )(x, agg, agg, w['pn_w1x'], w['pn_w1a'], w['pn_b1'], w['pn_w2'],
      w['pn_b2'], w['pn_w3'], w['pn_b3'], w['pn_g'], w['pn_bt'],
      w['wsn'], w['wrn'])


def _decoder(x, l8, p8, w):
    grid = NP // BLK_N
    return pl.pallas_call(
        _dec_body,
        grid=grid,
        in_specs=[
            _row_spec(BLK_N, LATENT), _row_spec(BLK_N, 8), _row_spec(BLK_N, 8),
            _w_spec(LATENT, LATENT), _w_spec(1, LATENT),
            _w_spec(LATENT, LATENT), _w_spec(1, LATENT),
            _w_spec(LATENT, 8), _w_spec(1, 8),
        ],
        out_specs=_row_spec(BLK_N, 8),
        out_shape=jax.ShapeDtypeStruct((NP, 8), _f32),
    )(x, l8, p8, w['d_w1'], w['d_b1'], w['d_w2'], w['d_b2'], w['d_w3p'],
      w['d_b3p'])


def _row(v):
    return v.reshape(1, -1)


def kernel(position_sequence, edge_index, particle_types, params):
    pos = position_sequence.astype(_f32)
    last = pos[:, -1]
    prev = pos[:, -2]

    posflat = jnp.pad(pos.reshape(N, SEQ * DIM), ((0, NP - N), (0, 0)))
    types2 = jnp.pad(particle_types.reshape(N, 1).astype(jnp.int32),
                     ((0, NP - N), (0, 0)))
    sidx = jnp.pad(edge_index[0], (0, EP - E)).reshape(NW, NB, EB)
    ridx = jnp.pad(edge_index[1], (0, EP - E)).reshape(NW, NB, EB)
    ridx_s = jnp.pad(edge_index[1], (0, EP - E)).reshape(NW, NB_S, EB_S)
    pos128 = jnp.pad(last, ((0, NP - N), (0, LATENT - DIM)))
    l8 = jnp.pad(last, ((0, NP - N), (0, 8 - DIM)))
    p8 = jnp.pad(prev, ((0, NP - N), (0, 8 - DIM)))
    zeros128 = jnp.zeros((EB_S, LATENT), _f32)

    ne = params['node_encoder']
    ee = params['edge_encoder']
    de = params['decoder']
    procs = params['processor']

    nW1 = ne['layers'][0][0]
    w = {
        'n_w1v': nW1[0:15], 'n_w1dl': nW1[15:18], 'n_w1du': nW1[18:21],
        'n_w1t': params['type_emb'] @ nW1[21:37],
        'n_b1': _row(ne['layers'][0][1]),
        'n_w2': ne['layers'][1][0], 'n_b2': _row(ne['layers'][1][1]),
        'n_w3': ne['layers'][2][0], 'n_b3': _row(ne['layers'][2][1]),
        'n_g': _row(ne['ln'][0]), 'n_bt': _row(ne['ln'][1]),
        'e_w1r': ee['layers'][0][0][0:3],
        'e_w1d': _row(ee['layers'][0][0][3]),
        'e_b1': _row(ee['layers'][0][1]),
        'e_w2': ee['layers'][1][0], 'e_b2': _row(ee['layers'][1][1]),
        'e_w3': ee['layers'][2][0], 'e_b3': _row(ee['layers'][2][1]),
        'e_g': _row(ee['ln'][0]), 'e_bt': _row(ee['ln'][1]),
        'd_w1': de['layers'][0][0], 'd_b1': _row(de['layers'][0][1]),
        'd_w2': de['layers'][1][0], 'd_b2': _row(de['layers'][1][1]),
        'd_w3p': jnp.pad(de['layers'][2][0], ((0, 0), (0, 8 - DIM))),
        'd_b3p': _row(jnp.pad(de['layers'][2][1], (0, 8 - DIM))),
    }

    def stk(f):
        return jnp.stack([f(p) for p in procs])

    ws_all = stk(lambda p: p['edge']['layers'][0][0][0:LATENT])
    wr_all = stk(lambda p: p['edge']['layers'][0][0][LATENT:2 * LATENT])
    w['ws0'] = ws_all[0]
    w['wr0'] = wr_all[0]

    wstack = {
        'pe_we': stk(lambda p: p['edge']['layers'][0][0][2 * LATENT:]),
        'pe_b1': stk(lambda p: _row(p['edge']['layers'][0][1])),
        'pe_w2': stk(lambda p: p['edge']['layers'][1][0]),
        'pe_b2': stk(lambda p: _row(p['edge']['layers'][1][1])),
        'pe_w3': stk(lambda p: p['edge']['layers'][2][0]),
        'pe_b3': stk(lambda p: _row(p['edge']['layers'][2][1])),
        'pe_g': stk(lambda p: _row(p['edge']['ln'][0])),
        'pe_bt': stk(lambda p: _row(p['edge']['ln'][1])),
        'pn_w1x': stk(lambda p: p['node']['layers'][0][0][0:LATENT]),
        'pn_w1a': stk(lambda p: p['node']['layers'][0][0][LATENT:]),
        'pn_b1': stk(lambda p: _row(p['node']['layers'][0][1])),
        'pn_w2': stk(lambda p: p['node']['layers'][1][0]),
        'pn_b2': stk(lambda p: _row(p['node']['layers'][1][1])),
        'pn_w3': stk(lambda p: p['node']['layers'][2][0]),
        'pn_b3': stk(lambda p: _row(p['node']['layers'][2][1])),
        'pn_g': stk(lambda p: _row(p['node']['ln'][0])),
        'pn_bt': stk(lambda p: _row(p['node']['ln'][1])),
        'wsn': jnp.roll(ws_all, -1, axis=0),
        'wrn': jnp.roll(wr_all, -1, axis=0),
    }

    x0, A0, B0 = _node_enc(posflat, types2, w)
    gps, gpr = _sc_gather2(pos128, pos128, sidx, ridx)
    e0 = _edge_enc(gps, gpr, w)

    def step(carry, ws):
        x, e, A, B = carry
        gA, gB = _sc_gather2(A, B, sidx, ridx)
        eu, en = _edge_proc(gA, gB, e, ws)
        agg = _sc_scatter(eu, ridx_s, zeros128)
        xn, An, Bn = _node_proc(x, agg, ws)
        return (xn, en, An, Bn), None

    (x, _, _, _), _ = lax.scan(step, (x0, e0, A0, B0), wstack)

    out8 = _decoder(x, l8, p8, w)
    return out8[:N, 0:DIM]


# speedup vs baseline: 1.2105x; 1.0197x over previous
import functools

import jax
import jax.numpy as jnp
from jax import lax
from jax.experimental import pallas as pl
from jax.experimental.pallas import tpu as pltpu
from jax.experimental.pallas import tpu_sc as plsc

N = 10000
E = 160000
DIM = 3
SEQ = 6
LATENT = 128
EMB = 16
NTYPES = 9
RADIUS = 0.015
VEL_MEAN = 0.0
VEL_STD = 1.0
ACC_MEAN = 0.0
ACC_STD = 1.0
CLAMP = 1.0
LOW = 0.1
HIGH = 0.9

NP = 10240
NW = 32
EB = 128
NB = 40
EP = NW * NB * EB
EPW = NB * EB
RPT = NP // 16

BLK_E = 1024
BLK_N = 1024

_f32 = jnp.float32


def _ln(x, g, b):
    mu = jnp.mean(x, axis=-1, keepdims=True)
    xc = x - mu
    var = jnp.mean(xc * xc, axis=-1, keepdims=True)
    return xc / jnp.sqrt(var + 1e-5) * g + b


def _sc_gather2(A, B, sidx, ridx):
    D = A.shape[1]
    dt = A.dtype
    K, G = (3, 1)
    RG = G * EB
    NGRP = NB // G
    NROUND = NGRP // K + (NGRP % K != 0)
    mesh = plsc.VectorSubcoreMesh(core_axis_name="c", subcore_axis_name="s")

    @functools.partial(
        pl.kernel,
        out_type=(jax.ShapeDtypeStruct((EP, D), dt),) * 2,
        mesh=mesh,
        scratch_types=[
            pltpu.VMEM((NB, EB), jnp.int32),
            pltpu.VMEM((NB, EB), jnp.int32),
            pltpu.VMEM((K, RG, D), dt),
            pltpu.VMEM((K, RG, D), dt),
        ] + [pltpu.SemaphoreType.DMA] * (4 * K),
    )
    def k(a_hbm, b_hbm, si_hbm, ri_hbm, ga_hbm, gb_hbm, si_v, ri_v, bufa, bufb,
          *sems):
        sga = sems[0:K]
        sgb = sems[K:2 * K]
        soa = sems[2 * K:3 * K]
        sob = sems[3 * K:4 * K]
        c = lax.axis_index("c")
        s = lax.axis_index("s")
        wid = s * 2 + c
        base = wid * EPW
        pltpu.sync_copy(si_hbm.at[wid], si_v)
        pltpu.sync_copy(ri_hbm.at[wid], ri_v)

        def fire(gj, b):
            for u in range(G):
                blk = gj * G + u
                pltpu.async_copy(a_hbm.at[si_v.at[blk]],
                                 bufa.at[b].at[pl.ds(u * EB, EB)], sga[b])
                pltpu.async_copy(b_hbm.at[ri_v.at[blk]],
                                 bufb.at[b].at[pl.ds(u * EB, EB)], sgb[b])

        for b in range(K):
            fire(b, b)

        def round_(g, carry):
            for b in range(K):
                gj = g * K + b

                @pl.when(gj < NGRP)
                def _():
                    dst = pl.ds(base + gj * RG, RG)
                    pltpu.make_async_copy(ga_hbm.at[dst], bufa.at[b],
                                          sga[b]).wait()
                    pltpu.make_async_copy(gb_hbm.at[dst], bufb.at[b],
                                          sgb[b]).wait()
                    oa = pltpu.async_copy(bufa.at[b], ga_hbm.at[dst], soa[b])
                    ob = pltpu.async_copy(bufb.at[b], gb_hbm.at[dst], sob[b])
                    oa.wait()
                    ob.wait()

                    @pl.when(gj + K < NGRP)
                    def _():
                        fire(gj + K, b)

            return carry

        lax.fori_loop(0, NROUND, round_, 0)

    return k(A, B, sidx, ridx)


EB_S = 128
NB_S = EPW // EB_S


def _sc_scatter(eu, ridx_s, zeros128):
    mesh = plsc.VectorSubcoreMesh(core_axis_name="c", subcore_axis_name="s")

    K = 2

    @functools.partial(
        pl.kernel,
        out_type=jax.ShapeDtypeStruct((2, NP, LATENT), _f32),
        mesh=mesh,
        scratch_types=[
            pltpu.VMEM((NB_S, EB_S), jnp.int32),
            pltpu.VMEM((K, EB_S, LATENT), _f32),
            pltpu.VMEM_SHARED((NP, LATENT), _f32),
        ] + [pltpu.SemaphoreType.DMA] * (2 * K),
    )
    def k(eu_hbm, ri_hbm, z_hbm, out_hbm, ri_v, buf, acc, *sems):
        sr = sems[0:K]
        ss = sems[K:2 * K]
        c = lax.axis_index("c")
        s = lax.axis_index("s")
        wid = s * 2 + c
        base = wid * EPW
        pltpu.sync_copy(ri_hbm.at[wid], ri_v)
        pltpu.sync_copy(z_hbm, buf.at[0].at[pl.ds(0, EB_S)])

        def zbody(t, carry):
            pltpu.sync_copy(buf.at[0].at[pl.ds(0, EB_S)],
                            acc.at[pl.ds(s * RPT + t * EB_S, EB_S)])
            return carry

        lax.fori_loop(0, RPT // EB_S, zbody, 0)
        plsc.subcore_barrier()

        for b in range(K):
            pltpu.async_copy(eu_hbm.at[pl.ds(base + b * EB_S, EB_S)],
                             buf.at[b], sr[b])

        NG = NB_S // K + (NB_S % K != 0)

        def group(g, carry):
            for b in range(K):
                j = g * K + b

                @pl.when(j < NB_S)
                def _():
                    pltpu.make_async_copy(
                        eu_hbm.at[pl.ds(base + j * EB_S, EB_S)], buf.at[b],
                        sr[b]).wait()
                    sc = pltpu.async_copy(buf.at[b], acc.at[ri_v.at[j]],
                                          ss[b], add=True)
                    sc.wait()

                    @pl.when(j + K < NB_S)
                    def _():
                        pltpu.async_copy(
                            eu_hbm.at[pl.ds(base + (j + K) * EB_S, EB_S)],
                            buf.at[b], sr[b])

            return carry

        lax.fori_loop(0, NG, group, 0)
        plsc.subcore_barrier()

        def obody(t, carry):
            r0 = s * RPT + t * EB_S
            pltpu.sync_copy(acc.at[pl.ds(r0, EB_S)],
                            buf.at[0].at[pl.ds(0, EB_S)])
            pltpu.sync_copy(buf.at[0].at[pl.ds(0, EB_S)],
                            out_hbm.at[c].at[pl.ds(r0, EB_S)])
            return carry

        lax.fori_loop(0, RPT // EB_S, obody, 0)

    return k(eu, ridx_s, zeros128)


def _node_enc_body(pf, tp, w1v, w1dl, w1du, w1t, b1, w2, b2, w3, b3, g, bt,
                   ws0, wr0, x_o, a_o, b_o):
    p = pf[...]
    vel = (p[:, 3:18] - p[:, 0:15] - VEL_MEAN) * (1.0 / VEL_STD)
    last = p[:, 15:18]
    ndl = jnp.clip((last - LOW) * (1.0 / RADIUS), -CLAMP, CLAMP)
    ndu = jnp.clip((HIGH - last) * (1.0 / RADIUS), -CLAMP, CLAMP)
    t = tp[...]
    oh = (t == lax.broadcasted_iota(jnp.int32, (t.shape[0], NTYPES), 1)
          ).astype(_f32)
    h1 = jnp.maximum(vel @ w1v[...] + ndl @ w1dl[...] + ndu @ w1du[...]
                     + oh @ w1t[...] + b1[...], 0.0)
    h2 = jnp.maximum(h1 @ w2[...] + b2[...], 0.0)
    x = _ln(h2 @ w3[...] + b3[...], g[...], bt[...])
    x_o[...] = x
    a_o[...] = x @ ws0[...]
    b_o[...] = x @ wr0[...]


def _edge_enc_body(ps, pr, w1r, w1d, b1, w2, b2, w3, b3, g, bt, e_o):
    rel = (ps[...][:, 0:3] + pr[...][:, 0:3]) * (1.0 / RADIUS)
    dist = jnp.sqrt(jnp.sum(rel * rel, axis=1, keepdims=True))
    h1 = jnp.maximum(rel @ w1r[...] + dist * w1d[...] + b1[...], 0.0)
    h2 = jnp.maximum(h1 @ w2[...] + b2[...], 0.0)
    e_o[...] = _ln(h2 @ w3[...] + b3[...], g[...], bt[...])


def _edge_proc_body(ga, gb, e_in, we, b1, w2, b2, w3, b3, g, bt, eu_o, en_o):
    i = pl.program_id(0)
    e = e_in[...]
    h1 = jnp.maximum(ga[...] + gb[...] + e @ we[...] + b1[...], 0.0)
    h2 = jnp.maximum(h1 @ w2[...] + b2[...], 0.0)
    u = _ln(h2 @ w3[...] + b3[...], g[...], bt[...])
    rows = i * BLK_E + lax.broadcasted_iota(jnp.int32, (BLK_E, 1), 0)
    u = jnp.where(rows < E, u, 0.0)
    eu_o[...] = u
    en_o[...] = e + u


def _node_proc_body(x_in, g0, g1, w1x, w1a, b1, w2, b2, w3, b3, g, bt,
                    wsn, wrn, x_o, a_o, b_o):
    x = x_in[...]
    agg = g0[...] + g1[...]
    h1 = jnp.maximum(x @ w1x[...] + agg @ w1a[...] + b1[...], 0.0)
    h2 = jnp.maximum(h1 @ w2[...] + b2[...], 0.0)
    u = _ln(h2 @ w3[...] + b3[...], g[...], bt[...])
    xn = x + u
    x_o[...] = xn
    a_o[...] = xn @ wsn[...]
    b_o[...] = xn @ wrn[...]


def _dec_body(x_in, l8, p8, w1, b1, w2, b2, w3p, b3p, o_ref):
    h1 = jnp.maximum(x_in[...] @ w1[...] + b1[...], 0.0)
    h2 = jnp.maximum(h1 @ w2[...] + b2[...], 0.0)
    acc8 = (h2 @ w3p[...] + b3p[...]) * ACC_STD + ACC_MEAN
    o_ref[...] = 2.0 * l8[...] - p8[...] + acc8


def _row_spec(blk, d):
    return pl.BlockSpec((blk, d), lambda i: (i, 0))


def _w_spec(d0, d1):
    return pl.BlockSpec((d0, d1), lambda i: (0, 0))


def _node_enc(posflat, types2, w):
    grid = NP // BLK_N
    out = pl.pallas_call(
        _node_enc_body,
        grid=grid,
        in_specs=[
            _row_spec(BLK_N, SEQ * DIM), _row_spec(BLK_N, 1),
            _w_spec(15, LATENT), _w_spec(3, LATENT), _w_spec(3, LATENT),
            _w_spec(NTYPES, LATENT), _w_spec(1, LATENT),
            _w_spec(LATENT, LATENT), _w_spec(1, LATENT),
            _w_spec(LATENT, LATENT), _w_spec(1, LATENT),
            _w_spec(1, LATENT), _w_spec(1, LATENT),
            _w_spec(LATENT, LATENT), _w_spec(LATENT, LATENT),
        ],
        out_specs=[_row_spec(BLK_N, LATENT)] * 3,
        out_shape=[jax.ShapeDtypeStruct((NP, LATENT), _f32)] * 3,
    )(posflat, types2, w['n_w1v'], w['n_w1dl'], w['n_w1du'], w['n_w1t'],
      w['n_b1'], w['n_w2'], w['n_b2'], w['n_w3'], w['n_b3'], w['n_g'],
      w['n_bt'], w['ws0'], w['wr0'])
    return out


def _edge_enc(gps, gpr, w):
    grid = EP // BLK_E
    return pl.pallas_call(
        _edge_enc_body,
        grid=grid,
        in_specs=[
            _row_spec(BLK_E, LATENT), _row_spec(BLK_E, LATENT),
            _w_spec(3, LATENT), _w_spec(1, LATENT), _w_spec(1, LATENT),
            _w_spec(LATENT, LATENT), _w_spec(1, LATENT),
            _w_spec(LATENT, LATENT), _w_spec(1, LATENT),
            _w_spec(1, LATENT), _w_spec(1, LATENT),
        ],
        out_specs=_row_spec(BLK_E, LATENT),
        out_shape=jax.ShapeDtypeStruct((EP, LATENT), _f32),
    )(gps, gpr, w['e_w1r'], w['e_w1d'], w['e_b1'], w['e_w2'], w['e_b2'],
      w['e_w3'], w['e_b3'], w['e_g'], w['e_bt'])


def _edge_proc(gA, gB, e, w):
    grid = EP // BLK_E
    return pl.pallas_call(
        _edge_proc_body,
        grid=grid,
        in_specs=[
            _row_spec(BLK_E, LATENT), _row_spec(BLK_E, LATENT),
            _row_spec(BLK_E, LATENT),
            _w_spec(LATENT, LATENT), _w_spec(1, LATENT),
            _w_spec(LATENT, LATENT), _w_spec(1, LATENT),
            _w_spec(LATENT, LATENT), _w_spec(1, LATENT),
            _w_spec(1, LATENT), _w_spec(1, LATENT),
        ],
        out_specs=[_row_spec(BLK_E, LATENT)] * 2,
        out_shape=[jax.ShapeDtypeStruct((EP, LATENT), _f32)] * 2,
    )(gA, gB, e, w['pe_we'], w['pe_b1'], w['pe_w2'], w['pe_b2'], w['pe_w3'],
      w['pe_b3'], w['pe_g'], w['pe_bt'])


def _node_proc(x, agg, w):
    grid = NP // BLK_N
    a3 = pl.BlockSpec((1, BLK_N, LATENT), lambda i: (0, i, 0))
    b3 = pl.BlockSpec((1, BLK_N, LATENT), lambda i: (1, i, 0))

    def body(x_in, agg_in0, agg_in1, *rest):
        _node_proc_body(x_in, agg_in0.at[0], agg_in1.at[0], *rest)

    return pl.pallas_call(
        body,
        grid=grid,
        in_specs=[
            _row_spec(BLK_N, LATENT), a3, b3,
            _w_spec(LATENT, LATENT), _w_spec(LATENT, LATENT),
            _w_spec(1, LATENT),
            _w_spec(LATENT, LATENT), _w_spec(1, LATENT),
            _w_spec(LATENT, LATENT), _w_spec(1, LATENT),
            _w_spec(1, LATENT), _w_spec(1, LATENT),
            _w_spec(LATENT, LATENT), _w_spec(LATENT, LATENT),
        ],
        out_specs=[_row_spec(BLK_N, LATENT)] * 3,
        out_shape=[jax.ShapeDtypeStruct((NP, LATENT), _f32)] * 3,
    )(x, agg, agg, w['pn_w1x'], w['pn_w1a'], w['pn_b1'], w['pn_w2'],
      w['pn_b2'], w['pn_w3'], w['pn_b3'], w['pn_g'], w['pn_bt'],
      w['wsn'], w['wrn'])


def _decoder(x, l8, p8, w):
    grid = NP // BLK_N
    return pl.pallas_call(
        _dec_body,
        grid=grid,
        in_specs=[
            _row_spec(BLK_N, LATENT), _row_spec(BLK_N, 8), _row_spec(BLK_N, 8),
            _w_spec(LATENT, LATENT), _w_spec(1, LATENT),
            _w_spec(LATENT, LATENT), _w_spec(1, LATENT),
            _w_spec(LATENT, 8), _w_spec(1, 8),
        ],
        out_specs=_row_spec(BLK_N, 8),
        out_shape=jax.ShapeDtypeStruct((NP, 8), _f32),
    )(x, l8, p8, w['d_w1'], w['d_b1'], w['d_w2'], w['d_b2'], w['d_w3p'],
      w['d_b3p'])


def _row(v):
    return v.reshape(1, -1)


def kernel(position_sequence, edge_index, particle_types, params):
    pos = position_sequence.astype(_f32)
    last = pos[:, -1]
    prev = pos[:, -2]

    posflat = jnp.pad(pos.reshape(N, SEQ * DIM), ((0, NP - N), (0, 0)))
    types2 = jnp.pad(particle_types.reshape(N, 1).astype(jnp.int32),
                     ((0, NP - N), (0, 0)))
    sidx = jnp.pad(edge_index[0], (0, EP - E)).reshape(NW, NB, EB)
    ridx = jnp.pad(edge_index[1], (0, EP - E)).reshape(NW, NB, EB)
    ridx_s = jnp.pad(edge_index[1], (0, EP - E)).reshape(NW, NB_S, EB_S)
    pos128 = jnp.pad(last, ((0, NP - N), (0, LATENT - DIM)))
    neg128 = jnp.pad(-last, ((0, NP - N), (0, LATENT - DIM)))
    l8 = jnp.pad(last, ((0, NP - N), (0, 8 - DIM)))
    p8 = jnp.pad(prev, ((0, NP - N), (0, 8 - DIM)))
    zeros128 = jnp.zeros((EB_S, LATENT), _f32)

    ne = params['node_encoder']
    ee = params['edge_encoder']
    de = params['decoder']
    procs = params['processor']

    nW1 = ne['layers'][0][0]
    w = {
        'n_w1v': nW1[0:15], 'n_w1dl': nW1[15:18], 'n_w1du': nW1[18:21],
        'n_w1t': params['type_emb'] @ nW1[21:37],
        'n_b1': _row(ne['layers'][0][1]),
        'n_w2': ne['layers'][1][0], 'n_b2': _row(ne['layers'][1][1]),
        'n_w3': ne['layers'][2][0], 'n_b3': _row(ne['layers'][2][1]),
        'n_g': _row(ne['ln'][0]), 'n_bt': _row(ne['ln'][1]),
        'e_w1r': ee['layers'][0][0][0:3],
        'e_w1d': _row(ee['layers'][0][0][3]),
        'e_b1': _row(ee['layers'][0][1]),
        'e_w2': ee['layers'][1][0], 'e_b2': _row(ee['layers'][1][1]),
        'e_w3': ee['layers'][2][0], 'e_b3': _row(ee['layers'][2][1]),
        'e_g': _row(ee['ln'][0]), 'e_bt': _row(ee['ln'][1]),
        'd_w1': de['layers'][0][0], 'd_b1': _row(de['layers'][0][1]),
        'd_w2': de['layers'][1][0], 'd_b2': _row(de['layers'][1][1]),
        'd_w3p': jnp.pad(de['layers'][2][0], ((0, 0), (0, 8 - DIM))),
        'd_b3p': _row(jnp.pad(de['layers'][2][1], (0, 8 - DIM))),
    }

    def stk(f):
        return jnp.stack([f(p) for p in procs])

    ws_all = stk(lambda p: p['edge']['layers'][0][0][0:LATENT])
    wr_all = stk(lambda p: p['edge']['layers'][0][0][LATENT:2 * LATENT])
    w['ws0'] = ws_all[0]
    w['wr0'] = wr_all[0]

    wstack = {
        'pe_we': stk(lambda p: p['edge']['layers'][0][0][2 * LATENT:]),
        'pe_b1': stk(lambda p: _row(p['edge']['layers'][0][1])),
        'pe_w2': stk(lambda p: p['edge']['layers'][1][0]),
        'pe_b2': stk(lambda p: _row(p['edge']['layers'][1][1])),
        'pe_w3': stk(lambda p: p['edge']['layers'][2][0]),
        'pe_b3': stk(lambda p: _row(p['edge']['layers'][2][1])),
        'pe_g': stk(lambda p: _row(p['edge']['ln'][0])),
        'pe_bt': stk(lambda p: _row(p['edge']['ln'][1])),
        'pn_w1x': stk(lambda p: p['node']['layers'][0][0][0:LATENT]),
        'pn_w1a': stk(lambda p: p['node']['layers'][0][0][LATENT:]),
        'pn_b1': stk(lambda p: _row(p['node']['layers'][0][1])),
        'pn_w2': stk(lambda p: p['node']['layers'][1][0]),
        'pn_b2': stk(lambda p: _row(p['node']['layers'][1][1])),
        'pn_w3': stk(lambda p: p['node']['layers'][2][0]),
        'pn_b3': stk(lambda p: _row(p['node']['layers'][2][1])),
        'pn_g': stk(lambda p: _row(p['node']['ln'][0])),
        'pn_bt': stk(lambda p: _row(p['node']['ln'][1])),
        'wsn': jnp.roll(ws_all, -1, axis=0),
        'wrn': jnp.roll(wr_all, -1, axis=0),
    }

    x0, A0, B0 = _node_enc(posflat, types2, w)
    gps, gpr = _sc_gather2(pos128, neg128, sidx, ridx)
    e0 = _edge_enc(gps, gpr, w)

    def step(carry, ws):
        x, e, A, B = carry
        gA, gB = _sc_gather2(A, B, sidx, ridx)
        eu, en = _edge_proc(gA, gB, e, ws)
        agg = _sc_scatter(eu, ridx_s, zeros128)
        xn, An, Bn = _node_proc(x, agg, ws)
        return (xn, en, An, Bn), None

    (x, _, _, _), _ = lax.scan(step, (x0, e0, A0, B0), wstack)

    out8 = _decoder(x, l8, p8, w)
    return out8[:N, 0:DIM]
